# Initial kernel scaffold; baseline (speedup 1.0000x reference)
import jax
import jax.numpy as jnp
from jax.experimental import pallas as pl


def kernel(hazard, survival, cut_points):
    raise NotImplementedError("write your pallas kernel here")


# speedup vs baseline: 1.4047x; 1.4047x over previous
import functools

import jax
import jax.numpy as jnp
from jax.experimental import pallas as pl
from jax.experimental.pallas import tpu as pltpu

GRID = 20


def _split_hi_lo(x):
    hi = x.astype(jnp.bfloat16)
    lo = (x - hi.astype(jnp.float32)).astype(jnp.bfloat16)
    return hi, lo


def _interp_kernel(haz_ref, surv_ref, cut_ref, ts_ref,
                   hstar_ref, satt_ref,
                   p0_ref, pd_ref, tsmT0_ref, rdT_ref, neg_ref):
    K = cut_ref.shape[1]
    M = ts_ref.shape[1]

    @pl.when(pl.program_id(0) == 0)
    def _build_tables():
        ts2 = ts_ref[:, :]
        cnt = jnp.zeros((1, M), jnp.int32)
        for k in range(K):
            cnt = cnt + (cut_ref[0, k] <= ts2).astype(jnp.int32)
        t0 = cnt - 1
        t1 = jnp.where(cnt == K, K - 1, cnt)
        T0 = jnp.zeros((1, M), jnp.float32)
        T1 = jnp.zeros((1, M), jnp.float32)
        for k in range(K):
            ck = cut_ref[0, k]
            T0 = jnp.where(t0 == k, ck, T0)
            T1 = jnp.where(t1 == k, ck, T1)
        dT = T1 - T0
        neg = (dT <= 0.0).astype(jnp.float32)
        dT_safe = jnp.where(dT <= 0.0, 1.0, dT)
        rdT_ref[:, :] = 1.0 / dT_safe
        tsmT0_ref[:, :] = ts2 - T0
        neg_ref[:, :] = neg
        ki = jax.lax.broadcasted_iota(jnp.int32, (K, M), 0)
        p0 = (ki == t0).astype(jnp.bfloat16)
        p1 = (ki == t1).astype(jnp.bfloat16)
        p0_ref[:, :] = p0
        pd_ref[:, :] = p0 - p1

    surv = surv_ref[:, :]
    haz = haz_ref[:, :]
    logs = jnp.log(1e-6 + surv)

    p0 = p0_ref[:, :]
    pd = pd_ref[:, :]

    def gather(x, p):
        hi, lo = _split_hi_lo(x)
        acc = jnp.dot(hi, p, preferred_element_type=jnp.float32)
        return acc + jnp.dot(lo, p, preferred_element_type=jnp.float32)

    S0 = gather(surv, p0)
    H0 = gather(haz, p0)
    Ld = gather(logs, pd)

    neg = neg_ref[:, :]
    hstar = jnp.where(neg > 0.0, H0, Ld * rdT_ref[:, :])
    hstar_ref[:, :] = hstar
    satt_ref[:, :] = S0 * jnp.exp(-tsmT0_ref[:, :] * hstar)


@jax.jit
def kernel(hazard, survival, cut_points):
    n, K = hazard.shape
    M = (K - 1) * GRID
    ts = jnp.linspace(cut_points[0], cut_points[-1], M)

    BN = 1024
    grid = (n // BN,)
    cut2 = cut_points.reshape(1, K)
    ts2 = ts.reshape(1, M)

    hstar, satt = pl.pallas_call(
        _interp_kernel,
        grid=grid,
        in_specs=[
            pl.BlockSpec((BN, K), lambda i: (i, 0)),
            pl.BlockSpec((BN, K), lambda i: (i, 0)),
            pl.BlockSpec((1, K), lambda i: (0, 0)),
            pl.BlockSpec((1, M), lambda i: (0, 0)),
        ],
        out_specs=[
            pl.BlockSpec((BN, M), lambda i: (i, 0)),
            pl.BlockSpec((BN, M), lambda i: (i, 0)),
        ],
        out_shape=[
            jax.ShapeDtypeStruct((n, M), jnp.float32),
            jax.ShapeDtypeStruct((n, M), jnp.float32),
        ],
        scratch_shapes=[
            pltpu.VMEM((K, M), jnp.bfloat16),
            pltpu.VMEM((K, M), jnp.bfloat16),
            pltpu.VMEM((1, M), jnp.float32),
            pltpu.VMEM((1, M), jnp.float32),
            pltpu.VMEM((1, M), jnp.float32),
        ],
    )(hazard, survival, cut2, ts2)
    return ts, hstar, satt


# speedup vs baseline: 1.5551x; 1.1070x over previous
import jax
import jax.numpy as jnp
from jax.experimental import pallas as pl
from jax.experimental.pallas import tpu as pltpu

GRID = 20


def _interp_kernel(haz_ref, surv_ref, cut_ref, ts_ref,
                   hstar_ref, satt_ref,
                   w1_ref, w2_ref, tsmT0_ref):
    K = cut_ref.shape[1]
    M = ts_ref.shape[1]

    @pl.when(pl.program_id(0) == 0)
    def _build_tables():
        ts2 = ts_ref[:, :]
        cnt = jnp.zeros((1, M), jnp.int32)
        for k in range(K):
            cnt = cnt + (cut_ref[0, k] <= ts2).astype(jnp.int32)
        t0 = cnt - 1
        t1 = jnp.where(cnt == K, K - 1, cnt)
        T0 = jnp.zeros((1, M), jnp.float32)
        T1 = jnp.zeros((1, M), jnp.float32)
        for k in range(K):
            ck = cut_ref[0, k]
            T0 = jnp.where(t0 == k, ck, T0)
            T1 = jnp.where(t1 == k, ck, T1)
        dT = T1 - T0
        neg = dT <= 0.0
        rdT = 1.0 / jnp.where(neg, 1.0, dT)
        tsmT0_ref[:, :] = ts2 - T0
        ki = jax.lax.broadcasted_iota(jnp.int32, (K, M), 0)
        p0 = (ki == t0).astype(jnp.float32)
        p1 = (ki == t1).astype(jnp.float32)
        pdr = (p0 - p1) * rdT
        p0n = p0 * jnp.where(neg, 1.0, 0.0)
        pdr_hi = pdr.astype(jnp.bfloat16)
        pdr_lo = (pdr - pdr_hi.astype(jnp.float32)).astype(jnp.bfloat16)
        p0n_bf = p0n.astype(jnp.bfloat16)
        p0_bf = p0.astype(jnp.bfloat16)
        w1_ref[0 * K:1 * K, :] = pdr_hi
        w1_ref[1 * K:2 * K, :] = pdr_hi
        w1_ref[2 * K:3 * K, :] = pdr_lo
        w1_ref[3 * K:4 * K, :] = p0n_bf
        w1_ref[4 * K:5 * K, :] = p0n_bf
        w2_ref[0 * K:1 * K, :] = p0_bf
        w2_ref[1 * K:2 * K, :] = p0_bf

    surv = surv_ref[:, :]
    haz = haz_ref[:, :]
    logs = jnp.log(1e-6 + surv)

    def split(x):
        hi = x.astype(jnp.bfloat16)
        lo = (x - hi.astype(jnp.float32)).astype(jnp.bfloat16)
        return hi, lo

    s_hi, s_lo = split(surv)
    h_hi, h_lo = split(haz)
    l_hi, l_lo = split(logs)

    lhs1 = jnp.concatenate([l_hi, l_lo, l_hi, h_hi, h_lo], axis=1)
    lhs2 = jnp.concatenate([s_hi, s_lo], axis=1)

    hstar = jnp.dot(lhs1, w1_ref[:, :], preferred_element_type=jnp.float32)
    S0 = jnp.dot(lhs2, w2_ref[:, :], preferred_element_type=jnp.float32)
    hstar_ref[:, :] = hstar
    satt_ref[:, :] = S0 * jnp.exp(-tsmT0_ref[:, :] * hstar)


@jax.jit
def kernel(hazard, survival, cut_points):
    n, K = hazard.shape
    M = (K - 1) * GRID
    ts = jnp.linspace(cut_points[0], cut_points[-1], M)

    BN = 1024
    grid = (n // BN,)
    cut2 = cut_points.reshape(1, K)
    ts2 = ts.reshape(1, M)

    hstar, satt = pl.pallas_call(
        _interp_kernel,
        grid=grid,
        in_specs=[
            pl.BlockSpec((BN, K), lambda i: (i, 0)),
            pl.BlockSpec((BN, K), lambda i: (i, 0)),
            pl.BlockSpec((1, K), lambda i: (0, 0)),
            pl.BlockSpec((1, M), lambda i: (0, 0)),
        ],
        out_specs=[
            pl.BlockSpec((BN, M), lambda i: (i, 0)),
            pl.BlockSpec((BN, M), lambda i: (i, 0)),
        ],
        out_shape=[
            jax.ShapeDtypeStruct((n, M), jnp.float32),
            jax.ShapeDtypeStruct((n, M), jnp.float32),
        ],
        scratch_shapes=[
            pltpu.VMEM((5 * K, M), jnp.bfloat16),
            pltpu.VMEM((2 * K, M), jnp.bfloat16),
            pltpu.VMEM((1, M), jnp.float32),
        ],
    )(hazard, survival, cut2, ts2)
    return ts, hstar, satt
